# baseline (speedup 1.0000x reference)
import functools

import jax
import jax.numpy as jnp
from jax.experimental import pallas as pl
from jax.experimental.pallas import tpu as pltpu
from jax.experimental.pallas import tpu_sc as plsc

KNN = 20
EPS = 1e-5
NEG = -3.0e38
NB_D = 512
NB_E = 2048
NB_T = 1024
_TW = 128


def _dot_bf(a, b):
    return jnp.dot(a.astype(jnp.bfloat16), b.astype(jnp.bfloat16),
                   preferred_element_type=jnp.float32)


def _lrelu(x):
    return jnp.where(x >= 0, x, 0.2 * x)


def _bn_apply(h, s_ref, g_ref, b_ref, cnt):
    mu = (s_ref[0, :] + s_ref[2, :]) / cnt
    var = (s_ref[1, :] + s_ref[3, :]) / cnt - mu * mu
    return (h - mu) / jnp.sqrt(var + EPS) * g_ref[0, :] + b_ref[0, :]


def _acc_sums(acc_ref, h):
    for r, blk in ((0, jnp.sum(h, axis=0, keepdims=True)),
                   (1, jnp.sum(h * h, axis=0, keepdims=True))):
        a = acc_ref[r:r + 1, :]
        t = a + blk
        acc_ref[r + 2:r + 3, :] += jnp.where(
            jnp.abs(a) >= jnp.abs(blk), (a - t) + blk, (blk - t) + a)
        acc_ref[r:r + 1, :] = t


def _knn_body(xrow_ref, xcol_ref, idx_ref, d_ref, *, n):
    b = pl.program_id(0)
    xb = xrow_ref[0]
    xc = xcol_ref[0]
    inner = _dot_bf(xb, xc)
    xxb = jnp.sum(xb * xb, axis=1, keepdims=True)
    xxf = jnp.sum(xc * xc, axis=0, keepdims=True)
    d_ref[...] = 2.0 * inner - xxb - xxf
    lane = jax.lax.broadcasted_iota(jnp.int32, (NB_D, n), 1)
    base = b * n
    for j in range(KNN):
        d = d_ref[...]
        m = jnp.max(d, axis=1, keepdims=True)
        am = jnp.min(jnp.where(d == m, lane, n), axis=1)
        idx_ref[0, j, :] = am + base
        d_ref[...] = jnp.where(lane == am[:, None], NEG, d)


def _knn(xrow, xcol):
    bsz, n, c = xrow.shape
    nblk = n // NB_D
    return pl.pallas_call(
        functools.partial(_knn_body, n=n),
        grid=(bsz, nblk),
        in_specs=[
            pl.BlockSpec((1, NB_D, c), lambda b, i: (b, i, 0)),
            pl.BlockSpec((1, c, n), lambda b, i: (b, 0, 0)),
        ],
        out_specs=pl.BlockSpec((1, KNN, NB_D), lambda b, i: (b, 0, i)),
        out_shape=jax.ShapeDtypeStruct((bsz, KNN, n), jnp.int32),
        scratch_shapes=[pltpu.VMEM((NB_D, n), jnp.float32)],
    )(xrow, xcol)


def _gather_rows(table, idx_flat):
    m = idx_flat.shape[1]
    dim = table.shape[1]
    gw = 128
    mesh = plsc.VectorSubcoreMesh(core_axis_name="c", subcore_axis_name="s")

    @functools.partial(
        pl.kernel,
        out_type=jax.ShapeDtypeStruct((m, dim), table.dtype),
        mesh=mesh)
    def kern(x_hbm, i_hbm, o_hbm):
        def body(i_vmem, o_vmem):
            pltpu.sync_copy(x_hbm.at[i_vmem.at[0]], o_vmem)

        pltpu.emit_pipeline(
            body,
            grid=(m // gw,),
            in_specs=[pl.BlockSpec((1, gw), index_map=lambda i: (0, i))],
            out_specs=[pl.BlockSpec((gw, dim), index_map=lambda i: (i, 0))],
            core_axis_name=("c", "s"),
            dimension_semantics=(pltpu.PARALLEL,),
        )(i_hbm, o_hbm)

    return kern(table, idx_flat)


def _edge_f(g_ref, x_ref, c):
    gj = g_ref[0, 0][:, 0:c]
    xi = x_ref[0][:, 0:c]
    return jnp.concatenate([gj - xi, xi], axis=1)


def _edge_stats_body(g_ref, x_ref, wt_ref, o_ref, acc_ref, *, c):
    first = ((pl.program_id(0) == 0) & (pl.program_id(1) == 0)
             & (pl.program_id(2) == 0))
    h = _dot_bf(_edge_f(g_ref, x_ref, c), wt_ref[...])

    @pl.when(first)
    def _():
        acc_ref[...] = jnp.zeros_like(acc_ref)

    _acc_sums(acc_ref, h)
    o_ref[...] = acc_ref[...]


def _edge_stats(g4, xrow, w1t):
    bsz, _, n, _ = g4.shape
    c = xrow.shape[2]
    dim = w1t.shape[1]
    nblk = n // NB_E
    return pl.pallas_call(
        functools.partial(_edge_stats_body, c=c),
        grid=(bsz, KNN, nblk),
        in_specs=[
            pl.BlockSpec((1, 1, NB_E, _TW), lambda b, j, i: (b, j, i, 0)),
            pl.BlockSpec((1, NB_E, c), lambda b, j, i: (b, i, 0)),
            pl.BlockSpec(w1t.shape, lambda b, j, i: (0, 0)),
        ],
        out_specs=pl.BlockSpec((8, dim), lambda b, j, i: (0, 0)),
        out_shape=jax.ShapeDtypeStruct((8, dim), jnp.float32),
        scratch_shapes=[pltpu.VMEM((8, dim), jnp.float32)],
    )(g4, xrow, w1t)


def _edge_main_body(g_ref, x_ref, w1t_ref, s1_ref, w2t_ref, g1_ref, b1_ref,
                    m2_ref, s2_ref, acc_ref, mx_ref, *, cnt, c):
    b, nb, j = pl.program_id(0), pl.program_id(1), pl.program_id(2)
    h1 = _dot_bf(_edge_f(g_ref, x_ref, c), w1t_ref[...])
    a1 = _lrelu(_bn_apply(h1, s1_ref, g1_ref, b1_ref, cnt))
    h2 = _dot_bf(a1, w2t_ref[...])

    @pl.when((b == 0) & (nb == 0) & (j == 0))
    def _():
        acc_ref[...] = jnp.zeros_like(acc_ref)

    _acc_sums(acc_ref, h2)
    s2_ref[...] = acc_ref[...]

    @pl.when(j == 0)
    def _():
        mx_ref[...] = h2

    @pl.when(j > 0)
    def _():
        mx_ref[...] = jnp.maximum(mx_ref[...], h2)

    m2_ref[0] = mx_ref[...]


def _edge_main(g4, xrow, w1t, s1, w2t, g1, b1):
    bsz, _, n, _ = g4.shape
    c = xrow.shape[2]
    dim = 64
    nblk = n // NB_E
    cnt = float(bsz * n * KNN)
    return pl.pallas_call(
        functools.partial(_edge_main_body, cnt=cnt, c=c),
        grid=(bsz, nblk, KNN),
        in_specs=[
            pl.BlockSpec((1, 1, NB_E, _TW), lambda b, i, j: (b, j, i, 0)),
            pl.BlockSpec((1, NB_E, c), lambda b, i, j: (b, i, 0)),
            pl.BlockSpec(w1t.shape, lambda b, i, j: (0, 0)),
            pl.BlockSpec((8, dim), lambda b, i, j: (0, 0)),
            pl.BlockSpec((dim, dim), lambda b, i, j: (0, 0)),
            pl.BlockSpec((1, dim), lambda b, i, j: (0, 0)),
            pl.BlockSpec((1, dim), lambda b, i, j: (0, 0)),
        ],
        out_specs=[
            pl.BlockSpec((1, NB_E, dim), lambda b, i, j: (b, i, 0)),
            pl.BlockSpec((8, dim), lambda b, i, j: (0, 0)),
        ],
        out_shape=[
            jax.ShapeDtypeStruct((bsz, n, dim), jnp.float32),
            jax.ShapeDtypeStruct((8, dim), jnp.float32),
        ],
        scratch_shapes=[pltpu.VMEM((8, dim), jnp.float32),
                        pltpu.VMEM((NB_E, dim), jnp.float32)],
    )(g4, xrow, w1t, s1, w2t, g1, b1)


def _edge_max_body(g_ref, x_ref, wt_ref, m2_ref, mx_ref, *, c):
    j = pl.program_id(2)
    h = _dot_bf(_edge_f(g_ref, x_ref, c), wt_ref[...])

    @pl.when(j == 0)
    def _():
        mx_ref[...] = h

    @pl.when(j > 0)
    def _():
        mx_ref[...] = jnp.maximum(mx_ref[...], h)

    m2_ref[0] = mx_ref[...]


def _edge_max(g4, xrow, w1t):
    bsz, _, n, _ = g4.shape
    c = xrow.shape[2]
    dim = w1t.shape[1]
    nblk = n // NB_E
    return pl.pallas_call(
        functools.partial(_edge_max_body, c=c),
        grid=(bsz, nblk, KNN),
        in_specs=[
            pl.BlockSpec((1, 1, NB_E, _TW), lambda b, i, j: (b, j, i, 0)),
            pl.BlockSpec((1, NB_E, c), lambda b, i, j: (b, i, 0)),
            pl.BlockSpec(w1t.shape, lambda b, i, j: (0, 0)),
        ],
        out_specs=pl.BlockSpec((1, NB_E, dim), lambda b, i, j: (b, i, 0)),
        out_shape=jax.ShapeDtypeStruct((bsz, n, dim), jnp.float32),
        scratch_shapes=[pltpu.VMEM((NB_E, dim), jnp.float32)],
    )(g4, xrow, w1t)


def _bn_act_body(m_ref, s_ref, g_ref, b_ref, x_ref, *, cnt):
    x_ref[0] = _lrelu(_bn_apply(m_ref[0], s_ref, g_ref, b_ref, cnt))


def _bn_act(m2, s, g, b, cnt):
    bsz, n, dim = m2.shape
    nblk = n // NB_E
    return pl.pallas_call(
        functools.partial(_bn_act_body, cnt=float(cnt)),
        grid=(bsz, nblk),
        in_specs=[
            pl.BlockSpec((1, NB_E, dim), lambda b, i: (b, i, 0)),
            pl.BlockSpec((8, dim), lambda b, i: (0, 0)),
            pl.BlockSpec((1, dim), lambda b, i: (0, 0)),
            pl.BlockSpec((1, dim), lambda b, i: (0, 0)),
        ],
        out_specs=pl.BlockSpec((1, NB_E, dim), lambda b, i: (b, i, 0)),
        out_shape=jax.ShapeDtypeStruct((bsz, n, dim), jnp.float32),
    )(m2, s, g, b)


def _cat_block(x1_ref, x2_ref, x3_ref):
    return jnp.concatenate([x1_ref[0], x2_ref[0], x3_ref[0]], axis=1)


def _tail_stats6_body(x1_ref, x2_ref, x3_ref, wt_ref, s_ref, acc_ref):
    first = (pl.program_id(0) == 0) & (pl.program_id(1) == 0)
    h = _dot_bf(_cat_block(x1_ref, x2_ref, x3_ref), wt_ref[...])

    @pl.when(first)
    def _():
        acc_ref[...] = jnp.zeros_like(acc_ref)

    _acc_sums(acc_ref, h)
    s_ref[...] = acc_ref[...]


def _tail_gf_body(x1_ref, x2_ref, x3_ref, wt_ref, s6_ref, g6_ref, b6_ref,
                  gf_ref, acc_ref, *, cnt, n):
    nb = pl.program_id(1)
    h = _dot_bf(_cat_block(x1_ref, x2_ref, x3_ref), wt_ref[...])
    vert = _lrelu(_bn_apply(h, s6_ref, g6_ref, b6_ref, cnt))

    @pl.when(nb == 0)
    def _():
        acc_ref[...] = jnp.zeros_like(acc_ref)

    acc_ref[0:1, :] += jnp.sum(vert, axis=0, keepdims=True)
    gf_ref[0] = acc_ref[...] * (1.0 / n)


def _tail_stats7_body(x1_ref, x2_ref, x3_ref, gf_ref, w7at_ref, w7bt_ref,
                      s_ref, acc_ref):
    first = (pl.program_id(0) == 0) & (pl.program_id(1) == 0)
    p = _dot_bf(gf_ref[0, 0:1, :], w7at_ref[...])
    q = _dot_bf(_cat_block(x1_ref, x2_ref, x3_ref), w7bt_ref[...])
    h = q + p

    @pl.when(first)
    def _():
        acc_ref[...] = jnp.zeros_like(acc_ref)

    _acc_sums(acc_ref, h)
    s_ref[...] = acc_ref[...]


def _tail_h8_body(x1_ref, x2_ref, x3_ref, gf_ref, w7at_ref, w7bt_ref,
                  s7_ref, g7_ref, b7_ref, w8t_ref, h8_ref, s8_ref,
                  acc_ref, *, cnt):
    first = (pl.program_id(0) == 0) & (pl.program_id(1) == 0)
    p = _dot_bf(gf_ref[0, 0:1, :], w7at_ref[...])
    q = _dot_bf(_cat_block(x1_ref, x2_ref, x3_ref), w7bt_ref[...])
    a7 = _lrelu(_bn_apply(q + p, s7_ref, g7_ref, b7_ref, cnt))
    h8 = _dot_bf(a7, w8t_ref[...])

    @pl.when(first)
    def _():
        acc_ref[...] = jnp.zeros_like(acc_ref)

    _acc_sums(acc_ref, h8)
    s8_ref[...] = acc_ref[...]
    h8_ref[0] = h8


def _tail_out_body(h8_ref, s8_ref, g8_ref, b8_ref, w9t_ref, o_ref, *, cnt):
    a8 = _lrelu(_bn_apply(h8_ref[0], s8_ref, g8_ref, b8_ref, cnt))
    o_ref[0] = _dot_bf(a8, w9t_ref[...])


def _edge_layer(xrow, xcol, w_first, w_second, g1, b1, g2, b2):
    bsz, n, c = xrow.shape
    idx = _knn(xrow, xcol)
    table = jnp.pad(xrow, ((0, 0), (0, 0), (0, _TW - c))).reshape(bsz * n, _TW)
    g = _gather_rows(table, idx.reshape(1, -1))
    g4 = g.reshape(bsz, KNN, n, _TW)
    w1t = w_first.T
    s1 = _edge_stats(g4, xrow, w1t)
    cnt = bsz * n * KNN
    if w_second is None:
        m = _edge_max(g4, xrow, w1t)
        return _bn_act(m, s1, g1, b1, cnt)
    m2, s2 = _edge_main(g4, xrow, w1t, s1, w_second.T, g1, b1)
    return _bn_act(m2, s2, g2, b2, cnt)


def kernel(x, params):
    p = params
    bsz, n, _ = x.shape
    nblk_t = n // NB_T
    x8 = jnp.pad(x, ((0, 0), (0, 0), (0, 5)))
    x8c = jnp.transpose(x8, (0, 2, 1))

    def row1(name):
        return p[name].reshape(1, -1)

    w1_8 = jnp.concatenate([jnp.pad(p['W1'][:, :3], ((0, 0), (0, 5))),
                            jnp.pad(p['W1'][:, 3:], ((0, 0), (0, 5)))], axis=1)
    x1 = _edge_layer(x8, x8c, w1_8,
                     p['W2'], row1('g1'), row1('b1'), row1('g2'), row1('b2'))
    x1c = jnp.transpose(x1, (0, 2, 1))
    x2 = _edge_layer(x1, x1c, p['W3'], p['W4'],
                     row1('g3'), row1('b3'), row1('g4'), row1('b4'))
    x2c = jnp.transpose(x2, (0, 2, 1))
    x3 = _edge_layer(x2, x2c, p['W5'], None,
                     row1('g5'), row1('b5'), None, None)

    cnt = float(bsz * n)
    w6t = p['W6'].T
    xspecs = [pl.BlockSpec((1, NB_T, 64), lambda b, i: (b, i, 0))] * 3
    stat_spec = pl.BlockSpec((8, 1024), lambda b, i: (0, 0))

    s6 = pl.pallas_call(
        _tail_stats6_body,
        grid=(bsz, nblk_t),
        in_specs=xspecs + [pl.BlockSpec((192, 1024), lambda b, i: (0, 0))],
        out_specs=stat_spec,
        out_shape=jax.ShapeDtypeStruct((8, 1024), jnp.float32),
        scratch_shapes=[pltpu.VMEM((8, 1024), jnp.float32)],
    )(x1, x2, x3, w6t)

    gf8 = pl.pallas_call(
        functools.partial(_tail_gf_body, cnt=cnt, n=float(n)),
        grid=(bsz, nblk_t),
        in_specs=xspecs + [
            pl.BlockSpec((192, 1024), lambda b, i: (0, 0)),
            stat_spec,
            pl.BlockSpec((1, 1024), lambda b, i: (0, 0)),
            pl.BlockSpec((1, 1024), lambda b, i: (0, 0)),
        ],
        out_specs=pl.BlockSpec((1, 8, 1024), lambda b, i: (b, 0, 0)),
        out_shape=jax.ShapeDtypeStruct((bsz, 8, 1024), jnp.float32),
        scratch_shapes=[pltpu.VMEM((8, 1024), jnp.float32)],
    )(x1, x2, x3, w6t, s6, row1('g6'), row1('b6'))

    w7at = p['W7'][:, :1024].T
    w7bt = p['W7'][:, 1024:].T
    gf_spec = pl.BlockSpec((1, 8, 1024), lambda b, i: (b, 0, 0))
    w7_specs = [pl.BlockSpec((1024, 512), lambda b, i: (0, 0)),
                pl.BlockSpec((192, 512), lambda b, i: (0, 0))]
    stat7_spec = pl.BlockSpec((8, 512), lambda b, i: (0, 0))

    s7 = pl.pallas_call(
        _tail_stats7_body,
        grid=(bsz, nblk_t),
        in_specs=xspecs + [gf_spec] + w7_specs,
        out_specs=stat7_spec,
        out_shape=jax.ShapeDtypeStruct((8, 512), jnp.float32),
        scratch_shapes=[pltpu.VMEM((8, 512), jnp.float32)],
    )(x1, x2, x3, gf8, w7at, w7bt)

    h8, s8 = pl.pallas_call(
        functools.partial(_tail_h8_body, cnt=cnt),
        grid=(bsz, nblk_t),
        in_specs=xspecs + [gf_spec] + w7_specs + [
            stat7_spec,
            pl.BlockSpec((1, 512), lambda b, i: (0, 0)),
            pl.BlockSpec((1, 512), lambda b, i: (0, 0)),
            pl.BlockSpec((512, 256), lambda b, i: (0, 0)),
        ],
        out_specs=[
            pl.BlockSpec((1, NB_T, 256), lambda b, i: (b, i, 0)),
            pl.BlockSpec((8, 256), lambda b, i: (0, 0)),
        ],
        out_shape=[
            jax.ShapeDtypeStruct((bsz, n, 256), jnp.float32),
            jax.ShapeDtypeStruct((8, 256), jnp.float32),
        ],
        scratch_shapes=[pltpu.VMEM((8, 256), jnp.float32)],
    )(x1, x2, x3, gf8, w7at, w7bt, s7, row1('g7'), row1('b7'), p['W8'].T)

    w9t = jnp.pad(p['W9'].T, ((0, 0), (0, 5)))
    out8 = pl.pallas_call(
        functools.partial(_tail_out_body, cnt=cnt),
        grid=(bsz, nblk_t),
        in_specs=[
            pl.BlockSpec((1, NB_T, 256), lambda b, i: (b, i, 0)),
            pl.BlockSpec((8, 256), lambda b, i: (0, 0)),
            pl.BlockSpec((1, 256), lambda b, i: (0, 0)),
            pl.BlockSpec((1, 256), lambda b, i: (0, 0)),
            pl.BlockSpec((256, 8), lambda b, i: (0, 0)),
        ],
        out_specs=pl.BlockSpec((1, NB_T, 8), lambda b, i: (b, i, 0)),
        out_shape=jax.ShapeDtypeStruct((bsz, n, 8), jnp.float32),
    )(h8, s8, row1('g8'), row1('b8'), w9t)

    return (gf8[:, 0, :], out8[:, :, :3])


# speedup vs baseline: 1.1002x; 1.1002x over previous
import functools

import jax
import jax.numpy as jnp
from jax.experimental import pallas as pl
from jax.experimental.pallas import tpu as pltpu
from jax.experimental.pallas import tpu_sc as plsc

KNN = 20
EPS = 1e-5
NEG = -3.0e38
NB_D = 512
NB_E = 4096
NB_T = 1024
_TW = 128


def _dot_bf(a, b):
    return jnp.dot(a.astype(jnp.bfloat16), b.astype(jnp.bfloat16),
                   preferred_element_type=jnp.float32)


def _lrelu(x):
    return jnp.where(x >= 0, x, 0.2 * x)


def _bn_apply(h, s_ref, g_ref, b_ref, cnt):
    mu = (s_ref[0, :] + s_ref[2, :]) / cnt
    var = (s_ref[1, :] + s_ref[3, :]) / cnt - mu * mu
    return (h - mu) / jnp.sqrt(var + EPS) * g_ref[0, :] + b_ref[0, :]


def _acc_sums(acc_ref, h):
    for r, blk in ((0, jnp.sum(h, axis=0, keepdims=True)),
                   (1, jnp.sum(h * h, axis=0, keepdims=True))):
        a = acc_ref[r:r + 1, :]
        t = a + blk
        acc_ref[r + 2:r + 3, :] += jnp.where(
            jnp.abs(a) >= jnp.abs(blk), (a - t) + blk, (blk - t) + a)
        acc_ref[r:r + 1, :] = t


def _knn_body(xrow_ref, xcol_ref, idx_ref, d_ref, *, n):
    b = pl.program_id(0)
    xb = xrow_ref[0]
    xc = xcol_ref[0]
    inner = _dot_bf(xb, xc)
    xxb = jnp.sum(xb * xb, axis=1, keepdims=True)
    xxf = jnp.sum(xc * xc, axis=0, keepdims=True)
    d_ref[...] = 2.0 * inner - xxb - xxf
    lane = jax.lax.broadcasted_iota(jnp.int32, (NB_D, n), 1)
    base = b * n
    for j in range(KNN):
        d = d_ref[...]
        am = jnp.argmax(d, axis=1).astype(jnp.int32)
        idx_ref[0, j, :] = am + base
        d_ref[...] = jnp.where(lane == am[:, None], NEG, d)


def _knn(xrow, xcol):
    bsz, n, c = xrow.shape
    nblk = n // NB_D
    return pl.pallas_call(
        functools.partial(_knn_body, n=n),
        grid=(bsz, nblk),
        in_specs=[
            pl.BlockSpec((1, NB_D, c), lambda b, i: (b, i, 0)),
            pl.BlockSpec((1, c, n), lambda b, i: (b, 0, 0)),
        ],
        out_specs=pl.BlockSpec((1, KNN, NB_D), lambda b, i: (b, 0, i)),
        out_shape=jax.ShapeDtypeStruct((bsz, KNN, n), jnp.int32),
        scratch_shapes=[pltpu.VMEM((NB_D, n), jnp.float32)],
        compiler_params=pltpu.CompilerParams(
            dimension_semantics=("parallel", "parallel")),
    )(xrow, xcol)


def _gather_rows(table, idx_flat):
    m = idx_flat.shape[1]
    dim = table.shape[1]
    gw = 128
    mesh = plsc.VectorSubcoreMesh(core_axis_name="c", subcore_axis_name="s")

    @functools.partial(
        pl.kernel,
        out_type=jax.ShapeDtypeStruct((m, dim), table.dtype),
        mesh=mesh)
    def kern(x_hbm, i_hbm, o_hbm):
        def body(i_vmem, o_vmem):
            pltpu.sync_copy(x_hbm.at[i_vmem.at[0]], o_vmem)

        pltpu.emit_pipeline(
            body,
            grid=(m // gw,),
            in_specs=[pl.BlockSpec((1, gw), index_map=lambda i: (0, i))],
            out_specs=[pl.BlockSpec((gw, dim), index_map=lambda i: (i, 0))],
            core_axis_name=("c", "s"),
            dimension_semantics=(pltpu.PARALLEL,),
        )(i_hbm, o_hbm)

    return kern(table, idx_flat)


def _edge_f(g_ref, x_ref, c):
    gj = g_ref[0, 0][:, 0:c]
    xi = x_ref[0][:, 0:c]
    return jnp.concatenate([gj - xi, xi], axis=1)


def _edge_stats_body(g_ref, x_ref, wt_ref, o_ref, acc_ref, *, c):
    first = ((pl.program_id(0) == 0) & (pl.program_id(1) == 0)
             & (pl.program_id(2) == 0))
    h = _dot_bf(_edge_f(g_ref, x_ref, c), wt_ref[...])

    @pl.when(first)
    def _():
        acc_ref[...] = jnp.zeros_like(acc_ref)

    _acc_sums(acc_ref, h)
    o_ref[...] = acc_ref[...]


def _edge_stats(g4, xrow, w1t):
    bsz, _, n, _ = g4.shape
    c = xrow.shape[2]
    dim = w1t.shape[1]
    nblk = n // NB_E
    return pl.pallas_call(
        functools.partial(_edge_stats_body, c=c),
        grid=(bsz, KNN, nblk),
        in_specs=[
            pl.BlockSpec((1, 1, NB_E, _TW), lambda b, j, i: (b, j, i, 0)),
            pl.BlockSpec((1, NB_E, c), lambda b, j, i: (b, i, 0)),
            pl.BlockSpec(w1t.shape, lambda b, j, i: (0, 0)),
        ],
        out_specs=pl.BlockSpec((8, dim), lambda b, j, i: (0, 0)),
        out_shape=jax.ShapeDtypeStruct((8, dim), jnp.float32),
        scratch_shapes=[pltpu.VMEM((8, dim), jnp.float32)],
    )(g4, xrow, w1t)


def _edge_main_body(g_ref, x_ref, w1t_ref, s1_ref, w2t_ref, g1_ref, b1_ref,
                    m2_ref, s2_ref, acc_ref, mx_ref, *, cnt, c):
    b, nb, j = pl.program_id(0), pl.program_id(1), pl.program_id(2)
    h1 = _dot_bf(_edge_f(g_ref, x_ref, c), w1t_ref[...])
    a1 = _lrelu(_bn_apply(h1, s1_ref, g1_ref, b1_ref, cnt))
    h2 = _dot_bf(a1, w2t_ref[...])

    @pl.when((b == 0) & (nb == 0) & (j == 0))
    def _():
        acc_ref[...] = jnp.zeros_like(acc_ref)

    _acc_sums(acc_ref, h2)
    s2_ref[...] = acc_ref[...]

    @pl.when(j == 0)
    def _():
        mx_ref[...] = h2

    @pl.when(j > 0)
    def _():
        mx_ref[...] = jnp.maximum(mx_ref[...], h2)

    m2_ref[0] = mx_ref[...]


def _edge_main(g4, xrow, w1t, s1, w2t, g1, b1):
    bsz, _, n, _ = g4.shape
    c = xrow.shape[2]
    dim = 64
    nblk = n // NB_E
    cnt = float(bsz * n * KNN)
    return pl.pallas_call(
        functools.partial(_edge_main_body, cnt=cnt, c=c),
        grid=(bsz, nblk, KNN),
        in_specs=[
            pl.BlockSpec((1, 1, NB_E, _TW), lambda b, i, j: (b, j, i, 0)),
            pl.BlockSpec((1, NB_E, c), lambda b, i, j: (b, i, 0)),
            pl.BlockSpec(w1t.shape, lambda b, i, j: (0, 0)),
            pl.BlockSpec((8, dim), lambda b, i, j: (0, 0)),
            pl.BlockSpec((dim, dim), lambda b, i, j: (0, 0)),
            pl.BlockSpec((1, dim), lambda b, i, j: (0, 0)),
            pl.BlockSpec((1, dim), lambda b, i, j: (0, 0)),
        ],
        out_specs=[
            pl.BlockSpec((1, NB_E, dim), lambda b, i, j: (b, i, 0)),
            pl.BlockSpec((8, dim), lambda b, i, j: (0, 0)),
        ],
        out_shape=[
            jax.ShapeDtypeStruct((bsz, n, dim), jnp.float32),
            jax.ShapeDtypeStruct((8, dim), jnp.float32),
        ],
        scratch_shapes=[pltpu.VMEM((8, dim), jnp.float32),
                        pltpu.VMEM((NB_E, dim), jnp.float32)],
    )(g4, xrow, w1t, s1, w2t, g1, b1)


def _edge_max_body(g_ref, x_ref, wt_ref, m2_ref, mx_ref, *, c):
    j = pl.program_id(2)
    h = _dot_bf(_edge_f(g_ref, x_ref, c), wt_ref[...])

    @pl.when(j == 0)
    def _():
        mx_ref[...] = h

    @pl.when(j > 0)
    def _():
        mx_ref[...] = jnp.maximum(mx_ref[...], h)

    m2_ref[0] = mx_ref[...]


def _edge_max(g4, xrow, w1t):
    bsz, _, n, _ = g4.shape
    c = xrow.shape[2]
    dim = w1t.shape[1]
    nblk = n // NB_E
    return pl.pallas_call(
        functools.partial(_edge_max_body, c=c),
        grid=(bsz, nblk, KNN),
        in_specs=[
            pl.BlockSpec((1, 1, NB_E, _TW), lambda b, i, j: (b, j, i, 0)),
            pl.BlockSpec((1, NB_E, c), lambda b, i, j: (b, i, 0)),
            pl.BlockSpec(w1t.shape, lambda b, i, j: (0, 0)),
        ],
        out_specs=pl.BlockSpec((1, NB_E, dim), lambda b, i, j: (b, i, 0)),
        out_shape=jax.ShapeDtypeStruct((bsz, n, dim), jnp.float32),
        scratch_shapes=[pltpu.VMEM((NB_E, dim), jnp.float32)],
        compiler_params=pltpu.CompilerParams(
            dimension_semantics=("parallel", "parallel", "arbitrary")),
    )(g4, xrow, w1t)


def _bn_act_body(m_ref, s_ref, g_ref, b_ref, x_ref, *, cnt):
    x_ref[0] = _lrelu(_bn_apply(m_ref[0], s_ref, g_ref, b_ref, cnt))


def _bn_act(m2, s, g, b, cnt):
    bsz, n, dim = m2.shape
    nblk = n // NB_E
    return pl.pallas_call(
        functools.partial(_bn_act_body, cnt=float(cnt)),
        grid=(bsz, nblk),
        in_specs=[
            pl.BlockSpec((1, NB_E, dim), lambda b, i: (b, i, 0)),
            pl.BlockSpec((8, dim), lambda b, i: (0, 0)),
            pl.BlockSpec((1, dim), lambda b, i: (0, 0)),
            pl.BlockSpec((1, dim), lambda b, i: (0, 0)),
        ],
        out_specs=pl.BlockSpec((1, NB_E, dim), lambda b, i: (b, i, 0)),
        out_shape=jax.ShapeDtypeStruct((bsz, n, dim), jnp.float32),
        compiler_params=pltpu.CompilerParams(
            dimension_semantics=("parallel", "parallel")),
    )(m2, s, g, b)


def _cat_block(x1_ref, x2_ref, x3_ref):
    return jnp.concatenate([x1_ref[0], x2_ref[0], x3_ref[0]], axis=1)


def _tail_stats6_body(x1_ref, x2_ref, x3_ref, wt_ref, s_ref, acc_ref):
    first = (pl.program_id(0) == 0) & (pl.program_id(1) == 0)
    h = _dot_bf(_cat_block(x1_ref, x2_ref, x3_ref), wt_ref[...])

    @pl.when(first)
    def _():
        acc_ref[...] = jnp.zeros_like(acc_ref)

    _acc_sums(acc_ref, h)
    s_ref[...] = acc_ref[...]


def _tail_gf_body(x1_ref, x2_ref, x3_ref, wt_ref, s6_ref, g6_ref, b6_ref,
                  gf_ref, acc_ref, *, cnt, n):
    nb = pl.program_id(1)
    h = _dot_bf(_cat_block(x1_ref, x2_ref, x3_ref), wt_ref[...])
    vert = _lrelu(_bn_apply(h, s6_ref, g6_ref, b6_ref, cnt))

    @pl.when(nb == 0)
    def _():
        acc_ref[...] = jnp.zeros_like(acc_ref)

    acc_ref[0:1, :] += jnp.sum(vert, axis=0, keepdims=True)
    gf_ref[0] = acc_ref[...] * (1.0 / n)


def _tail_stats7_body(x1_ref, x2_ref, x3_ref, gf_ref, w7at_ref, w7bt_ref,
                      s_ref, acc_ref):
    first = (pl.program_id(0) == 0) & (pl.program_id(1) == 0)
    p = _dot_bf(gf_ref[0, 0:1, :], w7at_ref[...])
    q = _dot_bf(_cat_block(x1_ref, x2_ref, x3_ref), w7bt_ref[...])
    h = q + p

    @pl.when(first)
    def _():
        acc_ref[...] = jnp.zeros_like(acc_ref)

    _acc_sums(acc_ref, h)
    s_ref[...] = acc_ref[...]


def _tail_h8_body(x1_ref, x2_ref, x3_ref, gf_ref, w7at_ref, w7bt_ref,
                  s7_ref, g7_ref, b7_ref, w8t_ref, h8_ref, s8_ref,
                  acc_ref, *, cnt):
    first = (pl.program_id(0) == 0) & (pl.program_id(1) == 0)
    p = _dot_bf(gf_ref[0, 0:1, :], w7at_ref[...])
    q = _dot_bf(_cat_block(x1_ref, x2_ref, x3_ref), w7bt_ref[...])
    a7 = _lrelu(_bn_apply(q + p, s7_ref, g7_ref, b7_ref, cnt))
    h8 = _dot_bf(a7, w8t_ref[...])

    @pl.when(first)
    def _():
        acc_ref[...] = jnp.zeros_like(acc_ref)

    _acc_sums(acc_ref, h8)
    s8_ref[...] = acc_ref[...]
    h8_ref[0] = h8


def _tail_out_body(h8_ref, s8_ref, g8_ref, b8_ref, w9t_ref, o_ref, *, cnt):
    a8 = _lrelu(_bn_apply(h8_ref[0], s8_ref, g8_ref, b8_ref, cnt))
    o_ref[0] = _dot_bf(a8, w9t_ref[...])


def _edge_layer(xrow, xcol, w_first, w_second, g1, b1, g2, b2):
    bsz, n, c = xrow.shape
    idx = _knn(xrow, xcol)
    table = jnp.pad(xrow, ((0, 0), (0, 0), (0, _TW - c))).reshape(bsz * n, _TW)
    g = _gather_rows(table, idx.reshape(1, -1))
    g4 = g.reshape(bsz, KNN, n, _TW)
    w1t = w_first.T
    s1 = _edge_stats(g4, xrow, w1t)
    cnt = bsz * n * KNN
    if w_second is None:
        m = _edge_max(g4, xrow, w1t)
        return _bn_act(m, s1, g1, b1, cnt)
    m2, s2 = _edge_main(g4, xrow, w1t, s1, w_second.T, g1, b1)
    return _bn_act(m2, s2, g2, b2, cnt)


def kernel(x, params):
    p = params
    bsz, n, _ = x.shape
    nblk_t = n // NB_T
    x8 = jnp.pad(x, ((0, 0), (0, 0), (0, 5)))
    x8c = jnp.transpose(x8, (0, 2, 1))

    def row1(name):
        return p[name].reshape(1, -1)

    w1_8 = jnp.concatenate([jnp.pad(p['W1'][:, :3], ((0, 0), (0, 5))),
                            jnp.pad(p['W1'][:, 3:], ((0, 0), (0, 5)))], axis=1)
    x1 = _edge_layer(x8, x8c, w1_8,
                     p['W2'], row1('g1'), row1('b1'), row1('g2'), row1('b2'))
    x1c = jnp.transpose(x1, (0, 2, 1))
    x2 = _edge_layer(x1, x1c, p['W3'], p['W4'],
                     row1('g3'), row1('b3'), row1('g4'), row1('b4'))
    x2c = jnp.transpose(x2, (0, 2, 1))
    x3 = _edge_layer(x2, x2c, p['W5'], None,
                     row1('g5'), row1('b5'), None, None)

    cnt = float(bsz * n)
    w6t = p['W6'].T
    xspecs = [pl.BlockSpec((1, NB_T, 64), lambda b, i: (b, i, 0))] * 3
    stat_spec = pl.BlockSpec((8, 1024), lambda b, i: (0, 0))

    s6 = pl.pallas_call(
        _tail_stats6_body,
        grid=(bsz, nblk_t),
        in_specs=xspecs + [pl.BlockSpec((192, 1024), lambda b, i: (0, 0))],
        out_specs=stat_spec,
        out_shape=jax.ShapeDtypeStruct((8, 1024), jnp.float32),
        scratch_shapes=[pltpu.VMEM((8, 1024), jnp.float32)],
    )(x1, x2, x3, w6t)

    gf8 = pl.pallas_call(
        functools.partial(_tail_gf_body, cnt=cnt, n=float(n)),
        grid=(bsz, nblk_t),
        in_specs=xspecs + [
            pl.BlockSpec((192, 1024), lambda b, i: (0, 0)),
            stat_spec,
            pl.BlockSpec((1, 1024), lambda b, i: (0, 0)),
            pl.BlockSpec((1, 1024), lambda b, i: (0, 0)),
        ],
        out_specs=pl.BlockSpec((1, 8, 1024), lambda b, i: (b, 0, 0)),
        out_shape=jax.ShapeDtypeStruct((bsz, 8, 1024), jnp.float32),
        scratch_shapes=[pltpu.VMEM((8, 1024), jnp.float32)],
    )(x1, x2, x3, w6t, s6, row1('g6'), row1('b6'))

    w7at = p['W7'][:, :1024].T
    w7bt = p['W7'][:, 1024:].T
    gf_spec = pl.BlockSpec((1, 8, 1024), lambda b, i: (b, 0, 0))
    w7_specs = [pl.BlockSpec((1024, 512), lambda b, i: (0, 0)),
                pl.BlockSpec((192, 512), lambda b, i: (0, 0))]
    stat7_spec = pl.BlockSpec((8, 512), lambda b, i: (0, 0))

    s7 = pl.pallas_call(
        _tail_stats7_body,
        grid=(bsz, nblk_t),
        in_specs=xspecs + [gf_spec] + w7_specs,
        out_specs=stat7_spec,
        out_shape=jax.ShapeDtypeStruct((8, 512), jnp.float32),
        scratch_shapes=[pltpu.VMEM((8, 512), jnp.float32)],
    )(x1, x2, x3, gf8, w7at, w7bt)

    h8, s8 = pl.pallas_call(
        functools.partial(_tail_h8_body, cnt=cnt),
        grid=(bsz, nblk_t),
        in_specs=xspecs + [gf_spec] + w7_specs + [
            stat7_spec,
            pl.BlockSpec((1, 512), lambda b, i: (0, 0)),
            pl.BlockSpec((1, 512), lambda b, i: (0, 0)),
            pl.BlockSpec((512, 256), lambda b, i: (0, 0)),
        ],
        out_specs=[
            pl.BlockSpec((1, NB_T, 256), lambda b, i: (b, i, 0)),
            pl.BlockSpec((8, 256), lambda b, i: (0, 0)),
        ],
        out_shape=[
            jax.ShapeDtypeStruct((bsz, n, 256), jnp.float32),
            jax.ShapeDtypeStruct((8, 256), jnp.float32),
        ],
        scratch_shapes=[pltpu.VMEM((8, 256), jnp.float32)],
    )(x1, x2, x3, gf8, w7at, w7bt, s7, row1('g7'), row1('b7'), p['W8'].T)

    w9t = jnp.pad(p['W9'].T, ((0, 0), (0, 5)))
    out8 = pl.pallas_call(
        functools.partial(_tail_out_body, cnt=cnt),
        grid=(bsz, nblk_t),
        in_specs=[
            pl.BlockSpec((1, NB_T, 256), lambda b, i: (b, i, 0)),
            pl.BlockSpec((8, 256), lambda b, i: (0, 0)),
            pl.BlockSpec((1, 256), lambda b, i: (0, 0)),
            pl.BlockSpec((1, 256), lambda b, i: (0, 0)),
            pl.BlockSpec((256, 8), lambda b, i: (0, 0)),
        ],
        out_specs=pl.BlockSpec((1, NB_T, 8), lambda b, i: (b, i, 0)),
        out_shape=jax.ShapeDtypeStruct((bsz, n, 8), jnp.float32),
    )(h8, s8, row1('g8'), row1('b8'), w9t)

    return (gf8[:, 0, :], out8[:, :, :3])


# speedup vs baseline: 1.1822x; 1.0745x over previous
import functools

import jax
import jax.numpy as jnp
from jax.experimental import pallas as pl
from jax.experimental.pallas import tpu as pltpu
from jax.experimental.pallas import tpu_sc as plsc

KNN = 20
EPS = 1e-5
NEG = -3.0e38
NB_D = 1024
NB_E = 4096
NB_T = 1024
_TW = 128


def _dot_bf(a, b):
    return jnp.dot(a.astype(jnp.bfloat16), b.astype(jnp.bfloat16),
                   preferred_element_type=jnp.float32)


def _lrelu(x):
    return jnp.where(x >= 0, x, 0.2 * x)


def _bn_apply(h, s_ref, g_ref, b_ref, cnt):
    mu = (s_ref[0, :] + s_ref[2, :]) / cnt
    var = (s_ref[1, :] + s_ref[3, :]) / cnt - mu * mu
    return (h - mu) / jnp.sqrt(var + EPS) * g_ref[0, :] + b_ref[0, :]


def _acc_sums(acc_ref, h):
    for r, blk in ((0, jnp.sum(h, axis=0, keepdims=True)),
                   (1, jnp.sum(h * h, axis=0, keepdims=True))):
        a = acc_ref[r:r + 1, :]
        t = a + blk
        acc_ref[r + 2:r + 3, :] += jnp.where(
            jnp.abs(a) >= jnp.abs(blk), (a - t) + blk, (blk - t) + a)
        acc_ref[r:r + 1, :] = t


def _knn_body(xrow_ref, xcol_ref, idx_ref, d_ref, *, n):
    b = pl.program_id(0)
    xb = xrow_ref[0]
    xc = xcol_ref[0]
    inner = _dot_bf(xb, xc)
    xxb = jnp.sum(xb * xb, axis=1, keepdims=True)
    xxf = jnp.sum(xc * xc, axis=0, keepdims=True)
    d_ref[...] = 2.0 * inner - xxb - xxf
    lane = jax.lax.broadcasted_iota(jnp.int32, (NB_D, n), 1)
    base = b * n
    am = None
    for j in range(KNN):
        d = d_ref[...]
        if am is not None:
            d = jnp.where(lane == am[:, None], NEG, d)
            d_ref[...] = d
        am = jnp.argmax(d, axis=1).astype(jnp.int32)
        idx_ref[0, j, :] = am + base


def _knn(xrow, xcol):
    bsz, n, c = xrow.shape
    nblk = n // NB_D
    return pl.pallas_call(
        functools.partial(_knn_body, n=n),
        grid=(bsz, nblk),
        in_specs=[
            pl.BlockSpec((1, NB_D, c), lambda b, i: (b, i, 0)),
            pl.BlockSpec((1, c, n), lambda b, i: (b, 0, 0)),
        ],
        out_specs=pl.BlockSpec((1, KNN, NB_D), lambda b, i: (b, 0, i)),
        out_shape=jax.ShapeDtypeStruct((bsz, KNN, n), jnp.int32),
        scratch_shapes=[pltpu.VMEM((NB_D, n), jnp.float32)],
        compiler_params=pltpu.CompilerParams(
            dimension_semantics=("parallel", "parallel")),
    )(xrow, xcol)


def _gather_rows(table, idx_flat):
    m = idx_flat.shape[1]
    dim = table.shape[1]
    gw = 128
    mesh = plsc.VectorSubcoreMesh(core_axis_name="c", subcore_axis_name="s")

    @functools.partial(
        pl.kernel,
        out_type=jax.ShapeDtypeStruct((m, dim), table.dtype),
        mesh=mesh)
    def kern(x_hbm, i_hbm, o_hbm):
        def body(i_vmem, o_vmem):
            pltpu.sync_copy(x_hbm.at[i_vmem.at[0]], o_vmem)

        pltpu.emit_pipeline(
            body,
            grid=(m // gw,),
            in_specs=[pl.BlockSpec((1, gw), index_map=lambda i: (0, i))],
            out_specs=[pl.BlockSpec((gw, dim), index_map=lambda i: (i, 0))],
            core_axis_name=("c", "s"),
            dimension_semantics=(pltpu.PARALLEL,),
        )(i_hbm, o_hbm)

    return kern(table, idx_flat)


def _edge_f(g_ref, x_ref, c):
    gj = g_ref[0, 0][:, 0:c]
    xi = x_ref[0][:, 0:c]
    return jnp.concatenate([gj - xi, xi], axis=1)


def _edge_stats_body(g_ref, x_ref, wt_ref, o_ref, acc_ref, *, c):
    first = ((pl.program_id(0) == 0) & (pl.program_id(1) == 0)
             & (pl.program_id(2) == 0))
    h = _dot_bf(_edge_f(g_ref, x_ref, c), wt_ref[...])

    @pl.when(first)
    def _():
        acc_ref[...] = jnp.zeros_like(acc_ref)

    _acc_sums(acc_ref, h)
    o_ref[...] = acc_ref[...]


def _edge_stats(g4, xrow, w1t):
    bsz, _, n, _ = g4.shape
    c = xrow.shape[2]
    dim = w1t.shape[1]
    nblk = n // NB_E
    return pl.pallas_call(
        functools.partial(_edge_stats_body, c=c),
        grid=(bsz, KNN, nblk),
        in_specs=[
            pl.BlockSpec((1, 1, NB_E, _TW), lambda b, j, i: (b, j, i, 0)),
            pl.BlockSpec((1, NB_E, c), lambda b, j, i: (b, i, 0)),
            pl.BlockSpec(w1t.shape, lambda b, j, i: (0, 0)),
        ],
        out_specs=pl.BlockSpec((8, dim), lambda b, j, i: (0, 0)),
        out_shape=jax.ShapeDtypeStruct((8, dim), jnp.float32),
        scratch_shapes=[pltpu.VMEM((8, dim), jnp.float32)],
    )(g4, xrow, w1t)


def _edge_main_body(g_ref, x_ref, w1t_ref, s1_ref, w2t_ref, g1_ref, b1_ref,
                    m2_ref, s2_ref, acc_ref, mx_ref, *, cnt, c):
    b, nb, j = pl.program_id(0), pl.program_id(1), pl.program_id(2)
    h1 = _dot_bf(_edge_f(g_ref, x_ref, c), w1t_ref[...])
    a1 = _lrelu(_bn_apply(h1, s1_ref, g1_ref, b1_ref, cnt))
    h2 = _dot_bf(a1, w2t_ref[...])

    @pl.when((b == 0) & (nb == 0) & (j == 0))
    def _():
        acc_ref[...] = jnp.zeros_like(acc_ref)

    _acc_sums(acc_ref, h2)
    s2_ref[...] = acc_ref[...]

    @pl.when(j == 0)
    def _():
        mx_ref[...] = h2

    @pl.when(j > 0)
    def _():
        mx_ref[...] = jnp.maximum(mx_ref[...], h2)

    m2_ref[0] = mx_ref[...]


def _edge_main(g4, xrow, w1t, s1, w2t, g1, b1):
    bsz, _, n, _ = g4.shape
    c = xrow.shape[2]
    dim = 64
    nblk = n // NB_E
    cnt = float(bsz * n * KNN)
    return pl.pallas_call(
        functools.partial(_edge_main_body, cnt=cnt, c=c),
        grid=(bsz, nblk, KNN),
        in_specs=[
            pl.BlockSpec((1, 1, NB_E, _TW), lambda b, i, j: (b, j, i, 0)),
            pl.BlockSpec((1, NB_E, c), lambda b, i, j: (b, i, 0)),
            pl.BlockSpec(w1t.shape, lambda b, i, j: (0, 0)),
            pl.BlockSpec((8, dim), lambda b, i, j: (0, 0)),
            pl.BlockSpec((dim, dim), lambda b, i, j: (0, 0)),
            pl.BlockSpec((1, dim), lambda b, i, j: (0, 0)),
            pl.BlockSpec((1, dim), lambda b, i, j: (0, 0)),
        ],
        out_specs=[
            pl.BlockSpec((1, NB_E, dim), lambda b, i, j: (b, i, 0)),
            pl.BlockSpec((8, dim), lambda b, i, j: (0, 0)),
        ],
        out_shape=[
            jax.ShapeDtypeStruct((bsz, n, dim), jnp.float32),
            jax.ShapeDtypeStruct((8, dim), jnp.float32),
        ],
        scratch_shapes=[pltpu.VMEM((8, dim), jnp.float32),
                        pltpu.VMEM((NB_E, dim), jnp.float32)],
    )(g4, xrow, w1t, s1, w2t, g1, b1)


def _edge_max_body(g_ref, x_ref, wt_ref, m2_ref, mx_ref, *, c):
    j = pl.program_id(2)
    h = _dot_bf(_edge_f(g_ref, x_ref, c), wt_ref[...])

    @pl.when(j == 0)
    def _():
        mx_ref[...] = h

    @pl.when(j > 0)
    def _():
        mx_ref[...] = jnp.maximum(mx_ref[...], h)

    m2_ref[0] = mx_ref[...]


def _edge_max(g4, xrow, w1t):
    bsz, _, n, _ = g4.shape
    c = xrow.shape[2]
    dim = w1t.shape[1]
    nblk = n // NB_E
    return pl.pallas_call(
        functools.partial(_edge_max_body, c=c),
        grid=(bsz, nblk, KNN),
        in_specs=[
            pl.BlockSpec((1, 1, NB_E, _TW), lambda b, i, j: (b, j, i, 0)),
            pl.BlockSpec((1, NB_E, c), lambda b, i, j: (b, i, 0)),
            pl.BlockSpec(w1t.shape, lambda b, i, j: (0, 0)),
        ],
        out_specs=pl.BlockSpec((1, NB_E, dim), lambda b, i, j: (b, i, 0)),
        out_shape=jax.ShapeDtypeStruct((bsz, n, dim), jnp.float32),
        scratch_shapes=[pltpu.VMEM((NB_E, dim), jnp.float32)],
        compiler_params=pltpu.CompilerParams(
            dimension_semantics=("parallel", "parallel", "arbitrary")),
    )(g4, xrow, w1t)


def _bn_act_body(m_ref, s_ref, g_ref, b_ref, x_ref, *, cnt):
    x_ref[0] = _lrelu(_bn_apply(m_ref[0], s_ref, g_ref, b_ref, cnt))


def _bn_act(m2, s, g, b, cnt):
    bsz, n, dim = m2.shape
    nblk = n // NB_E
    return pl.pallas_call(
        functools.partial(_bn_act_body, cnt=float(cnt)),
        grid=(bsz, nblk),
        in_specs=[
            pl.BlockSpec((1, NB_E, dim), lambda b, i: (b, i, 0)),
            pl.BlockSpec((8, dim), lambda b, i: (0, 0)),
            pl.BlockSpec((1, dim), lambda b, i: (0, 0)),
            pl.BlockSpec((1, dim), lambda b, i: (0, 0)),
        ],
        out_specs=pl.BlockSpec((1, NB_E, dim), lambda b, i: (b, i, 0)),
        out_shape=jax.ShapeDtypeStruct((bsz, n, dim), jnp.float32),
        compiler_params=pltpu.CompilerParams(
            dimension_semantics=("parallel", "parallel")),
    )(m2, s, g, b)


def _cat_block(x1_ref, x2_ref, x3_ref):
    return jnp.concatenate([x1_ref[0], x2_ref[0], x3_ref[0]], axis=1)


def _tail_stats6_body(x1_ref, x2_ref, x3_ref, wt_ref, s_ref, acc_ref):
    first = (pl.program_id(0) == 0) & (pl.program_id(1) == 0)
    h = _dot_bf(_cat_block(x1_ref, x2_ref, x3_ref), wt_ref[...])

    @pl.when(first)
    def _():
        acc_ref[...] = jnp.zeros_like(acc_ref)

    _acc_sums(acc_ref, h)
    s_ref[...] = acc_ref[...]


def _tail_gf_body(x1_ref, x2_ref, x3_ref, wt_ref, s6_ref, g6_ref, b6_ref,
                  gf_ref, acc_ref, *, cnt, n):
    nb = pl.program_id(1)
    h = _dot_bf(_cat_block(x1_ref, x2_ref, x3_ref), wt_ref[...])
    vert = _lrelu(_bn_apply(h, s6_ref, g6_ref, b6_ref, cnt))

    @pl.when(nb == 0)
    def _():
        acc_ref[...] = jnp.zeros_like(acc_ref)

    acc_ref[0:1, :] += jnp.sum(vert, axis=0, keepdims=True)
    gf_ref[0] = acc_ref[...] * (1.0 / n)


def _tail_stats7_body(x1_ref, x2_ref, x3_ref, gf_ref, w7at_ref, w7bt_ref,
                      s_ref, acc_ref):
    first = (pl.program_id(0) == 0) & (pl.program_id(1) == 0)
    p = _dot_bf(gf_ref[0, 0:1, :], w7at_ref[...])
    q = _dot_bf(_cat_block(x1_ref, x2_ref, x3_ref), w7bt_ref[...])
    h = q + p

    @pl.when(first)
    def _():
        acc_ref[...] = jnp.zeros_like(acc_ref)

    _acc_sums(acc_ref, h)
    s_ref[...] = acc_ref[...]


def _tail_h8_body(x1_ref, x2_ref, x3_ref, gf_ref, w7at_ref, w7bt_ref,
                  s7_ref, g7_ref, b7_ref, w8t_ref, h8_ref, s8_ref,
                  acc_ref, *, cnt):
    first = (pl.program_id(0) == 0) & (pl.program_id(1) == 0)
    p = _dot_bf(gf_ref[0, 0:1, :], w7at_ref[...])
    q = _dot_bf(_cat_block(x1_ref, x2_ref, x3_ref), w7bt_ref[...])
    a7 = _lrelu(_bn_apply(q + p, s7_ref, g7_ref, b7_ref, cnt))
    h8 = _dot_bf(a7, w8t_ref[...])

    @pl.when(first)
    def _():
        acc_ref[...] = jnp.zeros_like(acc_ref)

    _acc_sums(acc_ref, h8)
    s8_ref[...] = acc_ref[...]
    h8_ref[0] = h8


def _tail_out_body(h8_ref, s8_ref, g8_ref, b8_ref, w9t_ref, o_ref, *, cnt):
    a8 = _lrelu(_bn_apply(h8_ref[0], s8_ref, g8_ref, b8_ref, cnt))
    o_ref[0] = _dot_bf(a8, w9t_ref[...])


def _edge_layer(xrow, xcol, w_first, w_second, g1, b1, g2, b2):
    bsz, n, c = xrow.shape
    idx = _knn(xrow, xcol)
    table = jnp.pad(xrow, ((0, 0), (0, 0), (0, _TW - c))).reshape(bsz * n, _TW)
    g = _gather_rows(table, idx.reshape(1, -1))
    g4 = g.reshape(bsz, KNN, n, _TW)
    w1t = w_first.T
    s1 = _edge_stats(g4, xrow, w1t)
    cnt = bsz * n * KNN
    if w_second is None:
        m = _edge_max(g4, xrow, w1t)
        return _bn_act(m, s1, g1, b1, cnt)
    m2, s2 = _edge_main(g4, xrow, w1t, s1, w_second.T, g1, b1)
    return _bn_act(m2, s2, g2, b2, cnt)


def kernel(x, params):
    p = params
    bsz, n, _ = x.shape
    nblk_t = n // NB_T
    x8 = jnp.pad(x, ((0, 0), (0, 0), (0, 5)))
    x8c = jnp.transpose(x8, (0, 2, 1))

    def row1(name):
        return p[name].reshape(1, -1)

    w1_8 = jnp.concatenate([jnp.pad(p['W1'][:, :3], ((0, 0), (0, 5))),
                            jnp.pad(p['W1'][:, 3:], ((0, 0), (0, 5)))], axis=1)
    x1 = _edge_layer(x8, x8c, w1_8,
                     p['W2'], row1('g1'), row1('b1'), row1('g2'), row1('b2'))
    x1c = jnp.transpose(x1, (0, 2, 1))
    x2 = _edge_layer(x1, x1c, p['W3'], p['W4'],
                     row1('g3'), row1('b3'), row1('g4'), row1('b4'))
    x2c = jnp.transpose(x2, (0, 2, 1))
    x3 = _edge_layer(x2, x2c, p['W5'], None,
                     row1('g5'), row1('b5'), None, None)

    cnt = float(bsz * n)
    w6t = p['W6'].T
    xspecs = [pl.BlockSpec((1, NB_T, 64), lambda b, i: (b, i, 0))] * 3
    stat_spec = pl.BlockSpec((8, 1024), lambda b, i: (0, 0))

    s6 = pl.pallas_call(
        _tail_stats6_body,
        grid=(bsz, nblk_t),
        in_specs=xspecs + [pl.BlockSpec((192, 1024), lambda b, i: (0, 0))],
        out_specs=stat_spec,
        out_shape=jax.ShapeDtypeStruct((8, 1024), jnp.float32),
        scratch_shapes=[pltpu.VMEM((8, 1024), jnp.float32)],
    )(x1, x2, x3, w6t)

    gf8 = pl.pallas_call(
        functools.partial(_tail_gf_body, cnt=cnt, n=float(n)),
        grid=(bsz, nblk_t),
        in_specs=xspecs + [
            pl.BlockSpec((192, 1024), lambda b, i: (0, 0)),
            stat_spec,
            pl.BlockSpec((1, 1024), lambda b, i: (0, 0)),
            pl.BlockSpec((1, 1024), lambda b, i: (0, 0)),
        ],
        out_specs=pl.BlockSpec((1, 8, 1024), lambda b, i: (b, 0, 0)),
        out_shape=jax.ShapeDtypeStruct((bsz, 8, 1024), jnp.float32),
        scratch_shapes=[pltpu.VMEM((8, 1024), jnp.float32)],
    )(x1, x2, x3, w6t, s6, row1('g6'), row1('b6'))

    w7at = p['W7'][:, :1024].T
    w7bt = p['W7'][:, 1024:].T
    gf_spec = pl.BlockSpec((1, 8, 1024), lambda b, i: (b, 0, 0))
    w7_specs = [pl.BlockSpec((1024, 512), lambda b, i: (0, 0)),
                pl.BlockSpec((192, 512), lambda b, i: (0, 0))]
    stat7_spec = pl.BlockSpec((8, 512), lambda b, i: (0, 0))

    s7 = pl.pallas_call(
        _tail_stats7_body,
        grid=(bsz, nblk_t),
        in_specs=xspecs + [gf_spec] + w7_specs,
        out_specs=stat7_spec,
        out_shape=jax.ShapeDtypeStruct((8, 512), jnp.float32),
        scratch_shapes=[pltpu.VMEM((8, 512), jnp.float32)],
    )(x1, x2, x3, gf8, w7at, w7bt)

    h8, s8 = pl.pallas_call(
        functools.partial(_tail_h8_body, cnt=cnt),
        grid=(bsz, nblk_t),
        in_specs=xspecs + [gf_spec] + w7_specs + [
            stat7_spec,
            pl.BlockSpec((1, 512), lambda b, i: (0, 0)),
            pl.BlockSpec((1, 512), lambda b, i: (0, 0)),
            pl.BlockSpec((512, 256), lambda b, i: (0, 0)),
        ],
        out_specs=[
            pl.BlockSpec((1, NB_T, 256), lambda b, i: (b, i, 0)),
            pl.BlockSpec((8, 256), lambda b, i: (0, 0)),
        ],
        out_shape=[
            jax.ShapeDtypeStruct((bsz, n, 256), jnp.float32),
            jax.ShapeDtypeStruct((8, 256), jnp.float32),
        ],
        scratch_shapes=[pltpu.VMEM((8, 256), jnp.float32)],
    )(x1, x2, x3, gf8, w7at, w7bt, s7, row1('g7'), row1('b7'), p['W8'].T)

    w9t = jnp.pad(p['W9'].T, ((0, 0), (0, 5)))
    out8 = pl.pallas_call(
        functools.partial(_tail_out_body, cnt=cnt),
        grid=(bsz, nblk_t),
        in_specs=[
            pl.BlockSpec((1, NB_T, 256), lambda b, i: (b, i, 0)),
            pl.BlockSpec((8, 256), lambda b, i: (0, 0)),
            pl.BlockSpec((1, 256), lambda b, i: (0, 0)),
            pl.BlockSpec((1, 256), lambda b, i: (0, 0)),
            pl.BlockSpec((256, 8), lambda b, i: (0, 0)),
        ],
        out_specs=pl.BlockSpec((1, NB_T, 8), lambda b, i: (b, i, 0)),
        out_shape=jax.ShapeDtypeStruct((bsz, n, 8), jnp.float32),
    )(h8, s8, row1('g8'), row1('b8'), w9t)

    return (gf8[:, 0, :], out8[:, :, :3])
